# baseline (speedup 1.0000x reference)
import functools

import jax
import jax.numpy as jnp
from jax.experimental import pallas as pl
from jax.experimental.pallas import tpu as pltpu

_NUM_EXPERTS = 16
_CAPACITY = 256
_TOKENS = 8192
_DMODEL = 2048
_BLOCK_ROWS = 1024
_PHASE = _TOKENS // _BLOCK_ROWS
_GRID = _PHASE


def _body(xa_ref, xb_ref, w_ref, gates_ref, idx_ref, loss_ref,
          acc_ref):
    i = pl.program_id(0)

    @pl.when(i < _PHASE)
    def _gating():
        w = w_ref[...]
        half = _DMODEL // 2
        logits = (jnp.dot(xa_ref[...], w[:half],
                          preferred_element_type=jnp.float32) +
                  jnp.dot(xb_ref[...], w[half:],
                          preferred_element_type=jnp.float32))
        lt = logits.T

        m1 = jnp.max(lt, axis=0, keepdims=True)
        e = jnp.exp(lt - m1)
        s = jnp.sum(e, axis=0, keepdims=True)
        probs = e / s

        row = jax.lax.broadcasted_iota(jnp.int32, lt.shape, 0)
        i1 = jnp.min(jnp.where(lt == m1, row, _NUM_EXPERTS), axis=0,
                     keepdims=True)
        masked = jnp.where(row == i1, -jnp.inf, lt)
        m2 = jnp.max(masked, axis=0, keepdims=True)
        i2 = jnp.min(jnp.where(masked == m2, row, _NUM_EXPERTS), axis=0,
                     keepdims=True)

        g1 = 1.0 / (1.0 + jnp.exp(m2 - m1))
        g2 = 1.0 - g1

        gates_ref[...] = jnp.concatenate([g1, g2], axis=0).T
        idx_ref[...] = jnp.concatenate([i1, i2], axis=0).T

        part = jnp.sum(probs, axis=1, keepdims=True)

        @pl.when(i == 0)
        def _init():
            acc_ref[...] = part

        @pl.when(i > 0)
        def _acc():
            acc_ref[...] += part

        @pl.when(i == _PHASE - 1)
        def _loss():
            usage = acc_ref[...] / _TOKENS
            loss_ref[...] = jnp.sum(usage * jnp.log(usage * _NUM_EXPERTS),
                                    keepdims=True).reshape(1, 1)


@functools.partial(jax.jit)
def kernel(x, W):
    last = _PHASE - 1
    gates, idx, loss = pl.pallas_call(
        _body,
        grid=(_GRID,),
        in_specs=[
            pl.BlockSpec((_BLOCK_ROWS, _DMODEL // 2),
                         lambda i: (jnp.minimum(i, last), 0)),
            pl.BlockSpec((_BLOCK_ROWS, _DMODEL // 2),
                         lambda i: (jnp.minimum(i, last), 1)),
            pl.BlockSpec((_DMODEL, _NUM_EXPERTS), lambda i: (0, 0)),
        ],
        out_specs=[
            pl.BlockSpec((_BLOCK_ROWS, 2),
                         lambda i: (jnp.minimum(i, last), 0)),
            pl.BlockSpec((_BLOCK_ROWS, 2),
                         lambda i: (jnp.minimum(i, last), 0)),
            pl.BlockSpec((1, 1), lambda i: (0, 0)),
        ],
        out_shape=[
            jax.ShapeDtypeStruct((_TOKENS, 2), jnp.float32),
            jax.ShapeDtypeStruct((_TOKENS, 2), jnp.int32),
            jax.ShapeDtypeStruct((1, 1), jnp.float32),
        ],
        scratch_shapes=[
            pltpu.VMEM((_NUM_EXPERTS, 1), jnp.float32),
        ],
    )(x, x, W)
    disp = jnp.zeros((_TOKENS, _NUM_EXPERTS, _CAPACITY), jnp.float32)
    return gates, idx, disp, loss.reshape(())


# speedup vs baseline: 1.0211x; 1.0211x over previous
import functools

import jax
import jax.numpy as jnp
from jax.experimental import pallas as pl
from jax.experimental.pallas import tpu as pltpu

_NUM_EXPERTS = 16
_CAPACITY = 256
_TOKENS = 8192
_DMODEL = 2048
_READ_ROWS = 2048
_FILL_ROWS = 512
_PHASE = _TOKENS // _READ_ROWS
_FILL_STEPS = _TOKENS // _FILL_ROWS
_GRID = _PHASE + _FILL_STEPS


def _body(xa_ref, xb_ref, w_ref, gates_ref, idx_ref, disp_ref, loss_ref,
          acc_ref):
    i = pl.program_id(0)

    @pl.when((i == 0) | (i == _PHASE + 1))
    def _zero_disp():
        disp_ref[...] = jnp.zeros_like(disp_ref)

    @pl.when(i < _PHASE)
    def _gating():
        w = w_ref[...]
        half = _DMODEL // 2
        logits = (jnp.dot(xa_ref[...], w[:half],
                          preferred_element_type=jnp.float32) +
                  jnp.dot(xb_ref[...], w[half:],
                          preferred_element_type=jnp.float32))
        lt = logits.T

        m1 = jnp.max(lt, axis=0, keepdims=True)
        e = jnp.exp(lt - m1)
        s = jnp.sum(e, axis=0, keepdims=True)
        probs = e / s

        row = jax.lax.broadcasted_iota(jnp.int32, lt.shape, 0)
        i1 = jnp.min(jnp.where(lt == m1, row, _NUM_EXPERTS), axis=0,
                     keepdims=True)
        masked = jnp.where(row == i1, -jnp.inf, lt)
        m2 = jnp.max(masked, axis=0, keepdims=True)
        i2 = jnp.min(jnp.where(masked == m2, row, _NUM_EXPERTS), axis=0,
                     keepdims=True)

        g1 = 1.0 / (1.0 + jnp.exp(m2 - m1))
        g2 = 1.0 - g1

        gates_ref[...] = jnp.concatenate([g1, g2], axis=0).T
        idx_ref[...] = jnp.concatenate([i1, i2], axis=0).T

        part = jnp.sum(probs, axis=1, keepdims=True)

        @pl.when(i == 0)
        def _init():
            acc_ref[...] = part

        @pl.when(i > 0)
        def _acc():
            acc_ref[...] += part

        @pl.when(i == _PHASE - 1)
        def _loss():
            usage = acc_ref[...] / _TOKENS
            loss_ref[...] = jnp.sum(usage * jnp.log(usage * _NUM_EXPERTS),
                                    keepdims=True).reshape(1, 1)


@functools.partial(jax.jit)
def kernel(x, W):
    last = _PHASE - 1
    gates, idx, disp, loss = pl.pallas_call(
        _body,
        grid=(_GRID,),
        in_specs=[
            pl.BlockSpec((_READ_ROWS, _DMODEL // 2),
                         lambda i: (jnp.minimum(i, last), 0)),
            pl.BlockSpec((_READ_ROWS, _DMODEL // 2),
                         lambda i: (jnp.minimum(i, last), 1)),
            pl.BlockSpec((_DMODEL, _NUM_EXPERTS), lambda i: (0, 0)),
        ],
        out_specs=[
            pl.BlockSpec((_READ_ROWS, 2),
                         lambda i: (jnp.minimum(i, last), 0)),
            pl.BlockSpec((_READ_ROWS, 2),
                         lambda i: (jnp.minimum(i, last), 0)),
            pl.BlockSpec((_FILL_ROWS, _NUM_EXPERTS, _CAPACITY),
                         lambda i: (jnp.maximum(i - _PHASE, 0), 0, 0)),
            pl.BlockSpec((1, 1), lambda i: (0, 0)),
        ],
        out_shape=[
            jax.ShapeDtypeStruct((_TOKENS, 2), jnp.float32),
            jax.ShapeDtypeStruct((_TOKENS, 2), jnp.int32),
            jax.ShapeDtypeStruct((_TOKENS, _NUM_EXPERTS, _CAPACITY),
                                 jnp.float32),
            jax.ShapeDtypeStruct((1, 1), jnp.float32),
        ],
        scratch_shapes=[
            pltpu.VMEM((_NUM_EXPERTS, 1), jnp.float32),
        ],
    )(x, x, W)
    return gates, idx, disp, loss.reshape(())
